# baseline (speedup 1.0000x reference)
import functools

import jax
import jax.numpy as jnp
from jax import lax
from jax.experimental import pallas as pl
from jax.experimental.pallas import tpu as pltpu
from jax.experimental.pallas import tpu_sc as plsc

N = 10000
E = 320000
D = 128
NC = 2
NS = 16
NW = NC * NS
PER_W = E // NW
CH = 125
NCH = PER_W // CH
HF = NCH // 2
RPT = 624
RCH = 104
NRC = RPT // RCH
TAIL = N - NS * RPT

BLK = 1000
NBLK = N // BLK
NPAD = 10240


def _vsc_mesh():
    return plsc.VectorSubcoreMesh(
        core_axis_name="c", subcore_axis_name="s", num_cores=NC, num_subcores=NS
    )


def _deg_body(src_hbm, deg_hbm, src_v, ones_v, zero_v, acc):
    c = lax.axis_index("c")
    s = lax.axis_index("s")
    wid = s * NC + c
    soff = pl.multiple_of(wid * PER_W, 8)
    pltpu.sync_copy(src_hbm.at[pl.ds(soff, PER_W)], src_v)

    one16 = jnp.ones((16,), jnp.float32)

    def fill_ones(i, _):
        ones_v[pl.ds(i * 16, 16)] = one16
        return 0

    lax.fori_loop(0, PER_W // 16, fill_ones, 0)

    @pl.when(s == 0)
    def _():
        zero16 = jnp.zeros((16,), jnp.float32)

        def fill_zero(i, _):
            zero_v[pl.ds(i * 16, 16)] = zero16
            return 0

        lax.fori_loop(0, NPAD // 16, fill_zero, 0)
        pltpu.sync_copy(zero_v, acc)

    plsc.subcore_barrier()
    pltpu.sync_copy(ones_v, acc.at[src_v], add=True)
    plsc.subcore_barrier()

    @pl.when(s == 0)
    def _():
        off = pl.multiple_of(c * NPAD, NPAD)
        pltpu.sync_copy(acc, deg_hbm.at[pl.ds(off, NPAD)])


def _deg_call(src_flat):
    k = functools.partial(
        pl.kernel,
        out_type=jax.ShapeDtypeStruct((NC * NPAD,), jnp.float32),
        mesh=_vsc_mesh(),
        scratch_types=[
            pltpu.VMEM((PER_W,), jnp.int32),
            pltpu.VMEM((PER_W,), jnp.float32),
            pltpu.VMEM((NPAD,), jnp.float32),
            pltpu.VMEM_SHARED((NPAD,), jnp.float32),
        ],
    )(_deg_body)
    return k(src_flat)


def _mm0_body(x_ref, w_ref, y_ref):
    y_ref[...] = lax.dot_general(
        x_ref[...], w_ref[...], (((1,), (1,)), ((), ())),
        preferred_element_type=jnp.float32,
    )


def _mm0_call(x, W):
    return pl.pallas_call(
        _mm0_body,
        grid=(NBLK,),
        in_specs=[
            pl.BlockSpec((BLK, D), lambda i: (i, 0)),
            pl.BlockSpec((D, D), lambda i: (0, 0)),
        ],
        out_specs=pl.BlockSpec((BLK, D), lambda i: (i, 0)),
        out_shape=jax.ShapeDtypeStruct((N, D), jnp.float32),
    )(x, W)


def _scale_body(y0_ref, d0_ref, d1_ref, y_ref):
    deg = d0_ref[...] + d1_ref[...]
    norm = lax.rsqrt(jnp.maximum(deg, 1.0))
    y_ref[...] = y0_ref[...] * norm


def _scale_call(y0, deg_part):
    d0 = deg_part[:N].reshape(N, 1)
    d1 = deg_part[NPAD:NPAD + N].reshape(N, 1)
    return pl.pallas_call(
        _scale_body,
        grid=(NBLK,),
        in_specs=[
            pl.BlockSpec((BLK, D), lambda i: (i, 0)),
            pl.BlockSpec((BLK, 1), lambda i: (i, 0)),
            pl.BlockSpec((BLK, 1), lambda i: (i, 0)),
        ],
        out_specs=pl.BlockSpec((BLK, D), lambda i: (i, 0)),
        out_shape=jax.ShapeDtypeStruct((N, D), jnp.float32),
    )(y0, d0, d1)


def _agg_body(
    y_hbm, src_hbm, dst_hbm, part_hbm, src_v, dst_v, rows_a, rows_b, acc,
    gsem_a, gsem_b, ssem_a, ssem_b,
):
    c = lax.axis_index("c")
    s = lax.axis_index("s")
    wid = s * NC + c

    zero16 = jnp.zeros((16,), jnp.float32)

    def fill_zero(i, _):
        r = i // (D // 16)
        col = i % (D // 16)
        rows_a[r, pl.ds(col * 16, 16)] = zero16
        return 0

    lax.fori_loop(0, CH * (D // 16), fill_zero, 0)
    row0 = pl.multiple_of(s * RPT, 8)
    for kk in range(NRC):
        pltpu.async_copy(
            rows_a.at[pl.ds(0, RCH)], acc.at[pl.ds(row0 + kk * RCH, RCH)], gsem_a
        )

    @pl.when(s == NS - 1)
    def _():
        pltpu.async_copy(
            rows_a.at[pl.ds(0, TAIL)], acc.at[pl.ds(NS * RPT, TAIL)], gsem_b
        )

    for kk in range(NRC):
        pltpu.make_async_copy(
            rows_a.at[pl.ds(0, RCH)], acc.at[pl.ds(row0 + kk * RCH, RCH)], gsem_a
        ).wait()

    @pl.when(s == NS - 1)
    def _():
        pltpu.make_async_copy(
            rows_a.at[pl.ds(0, TAIL)], acc.at[pl.ds(NS * RPT, TAIL)], gsem_b
        ).wait()

    plsc.subcore_barrier()

    for h in range(2):
        hoff = pl.multiple_of(h * HF, 8)
        pltpu.sync_copy(src_hbm.at[wid, pl.ds(hoff, HF)], src_v)
        pltpu.sync_copy(dst_hbm.at[wid, pl.ds(hoff, HF)], dst_v)
        pltpu.async_copy(y_hbm.at[src_v.at[0]], rows_a, gsem_a)

        def body(i, _):
            j0 = 2 * i
            j1 = j0 + 1
            pltpu.async_copy(y_hbm.at[src_v.at[j1]], rows_b, gsem_b)
            pltpu.make_async_copy(y_hbm.at[src_v.at[j0]], rows_a, gsem_a).wait()
            pltpu.sync_copy(rows_a, acc.at[dst_v.at[j0]], add=True)

            @pl.when(j0 + 2 < HF)
            def _():
                pltpu.async_copy(y_hbm.at[src_v.at[j0 + 2]], rows_a, gsem_a)

            pltpu.make_async_copy(y_hbm.at[src_v.at[j1]], rows_b, gsem_b).wait()
            pltpu.sync_copy(rows_b, acc.at[dst_v.at[j1]], add=True)
            return 0

        lax.fori_loop(0, HF // 2, body, 0)
    plsc.subcore_barrier()

    for kk in range(NRC):
        pltpu.async_copy(
            acc.at[pl.ds(row0 + kk * RCH, RCH)],
            part_hbm.at[c, pl.ds(row0 + kk * RCH, RCH)],
            gsem_a,
        )

    @pl.when(s == NS - 1)
    def _():
        pltpu.async_copy(
            acc.at[pl.ds(NS * RPT, TAIL)], part_hbm.at[c, pl.ds(NS * RPT, TAIL)],
            gsem_b,
        )

    for kk in range(NRC):
        pltpu.make_async_copy(
            acc.at[pl.ds(row0 + kk * RCH, RCH)],
            part_hbm.at[c, pl.ds(row0 + kk * RCH, RCH)],
            gsem_a,
        ).wait()

    @pl.when(s == NS - 1)
    def _():
        pltpu.make_async_copy(
            acc.at[pl.ds(NS * RPT, TAIL)], part_hbm.at[c, pl.ds(NS * RPT, TAIL)],
            gsem_b,
        ).wait()


def _agg_call(y, src3, dst3):
    k = functools.partial(
        pl.kernel,
        out_type=jax.ShapeDtypeStruct((NC, N, D), jnp.float32),
        mesh=_vsc_mesh(),
        scratch_types=[
            pltpu.VMEM((HF, CH), jnp.int32),
            pltpu.VMEM((HF, CH), jnp.int32),
            pltpu.VMEM((CH, D), jnp.float32),
            pltpu.VMEM((CH, D), jnp.float32),
            pltpu.VMEM_SHARED((N, D), jnp.float32),
            pltpu.SemaphoreType.DMA,
            pltpu.SemaphoreType.DMA,
            pltpu.SemaphoreType.DMA,
            pltpu.SemaphoreType.DMA,
        ],
    )(_agg_body)
    return k(y, src3, dst3)


def _fin_body(p_ref, w_ref, b_ref, o_ref):
    bias2 = jnp.sum(w_ref[...], axis=1) + b_ref[0, :]
    o_ref[...] = p_ref[0] + p_ref[1] + bias2[None, :]


def _fin_call(part, W, b2):
    return pl.pallas_call(
        _fin_body,
        grid=(NBLK,),
        in_specs=[
            pl.BlockSpec((NC, BLK, D), lambda i: (0, i, 0)),
            pl.BlockSpec((D, D), lambda i: (0, 0)),
            pl.BlockSpec((1, D), lambda i: (0, 0)),
        ],
        out_specs=pl.BlockSpec((BLK, D), lambda i: (i, 0)),
        out_shape=jax.ShapeDtypeStruct((N, D), jnp.float32),
    )(part, W, b2)


def kernel(x, edge_index, W, b):
    src3 = edge_index[0].reshape(NW, NCH, CH)
    dst3 = edge_index[1].reshape(NW, NCH, CH)
    y0 = _mm0_call(x, W)
    deg_part = _deg_call(edge_index[0])
    y = _scale_call(y0, deg_part)
    part = _agg_call(y, src3, dst3)
    return _fin_call(part, W, b.reshape(1, D))


# speedup vs baseline: 1.0248x; 1.0248x over previous
import functools

import jax
import jax.numpy as jnp
from jax import lax
from jax.experimental import pallas as pl
from jax.experimental.pallas import tpu as pltpu
from jax.experimental.pallas import tpu_sc as plsc

N = 10000
E = 320000
D = 128
NC = 2
NS = 16
NW = NC * NS
PER_W = E // NW
CH = 125
NCH = PER_W // CH
HF = NCH // 2
RPT = 624
RCH = 104
NRC = RPT // RCH
TAIL = N - NS * RPT

BLK = 1000
NBLK = N // BLK
NPAD = 10240


def _vsc_mesh():
    return plsc.VectorSubcoreMesh(
        core_axis_name="c", subcore_axis_name="s", num_cores=NC, num_subcores=NS
    )


def _deg_body(src_hbm, deg_hbm, src_v, ones_v, zero_v, acc):
    c = lax.axis_index("c")
    s = lax.axis_index("s")
    wid = s * NC + c
    soff = pl.multiple_of(wid * PER_W, 8)
    pltpu.sync_copy(src_hbm.at[pl.ds(soff, PER_W)], src_v)

    one16 = jnp.ones((16,), jnp.float32)

    def fill_ones(i, _):
        ones_v[pl.ds(i * 16, 16)] = one16
        return 0

    lax.fori_loop(0, PER_W // 16, fill_ones, 0)

    @pl.when(s == 0)
    def _():
        zero16 = jnp.zeros((16,), jnp.float32)

        def fill_zero(i, _):
            zero_v[pl.ds(i * 16, 16)] = zero16
            return 0

        lax.fori_loop(0, NPAD // 16, fill_zero, 0)
        pltpu.sync_copy(zero_v, acc)

    plsc.subcore_barrier()
    pltpu.sync_copy(ones_v, acc.at[src_v], add=True)
    plsc.subcore_barrier()

    @pl.when(s == 0)
    def _():
        off = pl.multiple_of(c * NPAD, NPAD)
        pltpu.sync_copy(acc, deg_hbm.at[pl.ds(off, NPAD)])


def _deg_call(src_flat):
    k = functools.partial(
        pl.kernel,
        out_type=jax.ShapeDtypeStruct((NC * NPAD,), jnp.float32),
        mesh=_vsc_mesh(),
        scratch_types=[
            pltpu.VMEM((PER_W,), jnp.int32),
            pltpu.VMEM((PER_W,), jnp.float32),
            pltpu.VMEM((NPAD,), jnp.float32),
            pltpu.VMEM_SHARED((NPAD,), jnp.float32),
        ],
    )(_deg_body)
    return k(src_flat)


def _mm_body(x_ref, d0_ref, d1_ref, w_ref, y_ref):
    deg = d0_ref[...] + d1_ref[...]
    norm = lax.rsqrt(jnp.maximum(deg, 1.0))
    xs = x_ref[...] * norm
    y_ref[...] = lax.dot_general(
        xs, w_ref[...], (((1,), (1,)), ((), ())),
        preferred_element_type=jnp.float32,
    )


def _mm_call(x, deg_part, W):
    d0 = deg_part[:N].reshape(N, 1)
    d1 = deg_part[NPAD:NPAD + N].reshape(N, 1)
    return pl.pallas_call(
        _mm_body,
        grid=(NBLK,),
        in_specs=[
            pl.BlockSpec((BLK, D), lambda i: (i, 0)),
            pl.BlockSpec((BLK, 1), lambda i: (i, 0)),
            pl.BlockSpec((BLK, 1), lambda i: (i, 0)),
            pl.BlockSpec((D, D), lambda i: (0, 0)),
        ],
        out_specs=pl.BlockSpec((BLK, D), lambda i: (i, 0)),
        out_shape=jax.ShapeDtypeStruct((N, D), jnp.float32),
    )(x, d0, d1, W)


def _agg_body(
    y_hbm, src_hbm, dst_hbm, part_hbm, src_v, dst_v, rows_a, rows_b, acc,
    gsem_a, gsem_b, ssem_a, ssem_b,
):
    c = lax.axis_index("c")
    s = lax.axis_index("s")
    wid = s * NC + c

    zero16 = jnp.zeros((16,), jnp.float32)

    def fill_zero(i, _):
        r = i // (D // 16)
        col = i % (D // 16)
        rows_a[r, pl.ds(col * 16, 16)] = zero16
        return 0

    lax.fori_loop(0, CH * (D // 16), fill_zero, 0)
    row0 = pl.multiple_of(s * RPT, 8)
    for kk in range(NRC):
        pltpu.async_copy(
            rows_a.at[pl.ds(0, RCH)], acc.at[pl.ds(row0 + kk * RCH, RCH)], gsem_a
        )

    @pl.when(s == NS - 1)
    def _():
        pltpu.async_copy(
            rows_a.at[pl.ds(0, TAIL)], acc.at[pl.ds(NS * RPT, TAIL)], gsem_b
        )

    for kk in range(NRC):
        pltpu.make_async_copy(
            rows_a.at[pl.ds(0, RCH)], acc.at[pl.ds(row0 + kk * RCH, RCH)], gsem_a
        ).wait()

    @pl.when(s == NS - 1)
    def _():
        pltpu.make_async_copy(
            rows_a.at[pl.ds(0, TAIL)], acc.at[pl.ds(NS * RPT, TAIL)], gsem_b
        ).wait()

    plsc.subcore_barrier()

    for h in range(2):
        hoff = pl.multiple_of(h * HF, 8)
        pltpu.sync_copy(src_hbm.at[wid, pl.ds(hoff, HF)], src_v)
        pltpu.sync_copy(dst_hbm.at[wid, pl.ds(hoff, HF)], dst_v)
        pltpu.async_copy(y_hbm.at[src_v.at[0]], rows_a, gsem_a)

        def body(i, _):
            j0 = 2 * i
            j1 = j0 + 1
            pltpu.async_copy(y_hbm.at[src_v.at[j1]], rows_b, gsem_b)
            pltpu.make_async_copy(y_hbm.at[src_v.at[j0]], rows_a, gsem_a).wait()
            pltpu.sync_copy(rows_a, acc.at[dst_v.at[j0]], add=True)

            @pl.when(j0 + 2 < HF)
            def _():
                pltpu.async_copy(y_hbm.at[src_v.at[j0 + 2]], rows_a, gsem_a)

            pltpu.make_async_copy(y_hbm.at[src_v.at[j1]], rows_b, gsem_b).wait()
            pltpu.sync_copy(rows_b, acc.at[dst_v.at[j1]], add=True)
            return 0

        lax.fori_loop(0, HF // 2, body, 0)
    plsc.subcore_barrier()

    for kk in range(NRC):
        pltpu.async_copy(
            acc.at[pl.ds(row0 + kk * RCH, RCH)],
            part_hbm.at[c, pl.ds(row0 + kk * RCH, RCH)],
            gsem_a,
        )

    @pl.when(s == NS - 1)
    def _():
        pltpu.async_copy(
            acc.at[pl.ds(NS * RPT, TAIL)], part_hbm.at[c, pl.ds(NS * RPT, TAIL)],
            gsem_b,
        )

    for kk in range(NRC):
        pltpu.make_async_copy(
            acc.at[pl.ds(row0 + kk * RCH, RCH)],
            part_hbm.at[c, pl.ds(row0 + kk * RCH, RCH)],
            gsem_a,
        ).wait()

    @pl.when(s == NS - 1)
    def _():
        pltpu.make_async_copy(
            acc.at[pl.ds(NS * RPT, TAIL)], part_hbm.at[c, pl.ds(NS * RPT, TAIL)],
            gsem_b,
        ).wait()


def _agg_call(y, src3, dst3):
    k = functools.partial(
        pl.kernel,
        out_type=jax.ShapeDtypeStruct((NC, N, D), jnp.float32),
        mesh=_vsc_mesh(),
        scratch_types=[
            pltpu.VMEM((HF, CH), jnp.int32),
            pltpu.VMEM((HF, CH), jnp.int32),
            pltpu.VMEM((CH, D), jnp.float32),
            pltpu.VMEM((CH, D), jnp.float32),
            pltpu.VMEM_SHARED((N, D), jnp.float32),
            pltpu.SemaphoreType.DMA,
            pltpu.SemaphoreType.DMA,
            pltpu.SemaphoreType.DMA,
            pltpu.SemaphoreType.DMA,
        ],
    )(_agg_body)
    return k(y, src3, dst3)


def _fin_body(p_ref, w_ref, b_ref, o_ref):
    bias2 = jnp.sum(w_ref[...], axis=1) + b_ref[0, :]
    o_ref[...] = p_ref[0] + p_ref[1] + bias2[None, :]


def _fin_call(part, W, b2):
    return pl.pallas_call(
        _fin_body,
        grid=(NBLK,),
        in_specs=[
            pl.BlockSpec((NC, BLK, D), lambda i: (0, i, 0)),
            pl.BlockSpec((D, D), lambda i: (0, 0)),
            pl.BlockSpec((1, D), lambda i: (0, 0)),
        ],
        out_specs=pl.BlockSpec((BLK, D), lambda i: (i, 0)),
        out_shape=jax.ShapeDtypeStruct((N, D), jnp.float32),
    )(part, W, b2)


def kernel(x, edge_index, W, b):
    src3 = edge_index[0].reshape(NW, NCH, CH)
    dst3 = edge_index[1].reshape(NW, NCH, CH)
    deg_part = _deg_call(edge_index[0])
    y = _mm_call(x, deg_part, W)
    part = _agg_call(y, src3, dst3)
    return _fin_call(part, W, b.reshape(1, D))


# speedup vs baseline: 1.0525x; 1.0270x over previous
import functools

import jax
import jax.numpy as jnp
from jax import lax
from jax.experimental import pallas as pl
from jax.experimental.pallas import tpu as pltpu
from jax.experimental.pallas import tpu_sc as plsc

N = 10000
E = 320000
D = 128
NC = 2
NS = 16
NW = NC * NS
PER_W = E // NW
CH = 100
NCH = PER_W // CH
QF = NCH // 4
RPT = 624
RCH = 48
NRC = RPT // RCH
TAIL = N - NS * RPT

BLK = 1000
NBLK = N // BLK
NPAD = 10240


def _vsc_mesh():
    return plsc.VectorSubcoreMesh(
        core_axis_name="c", subcore_axis_name="s", num_cores=NC, num_subcores=NS
    )


def _deg_body(src_hbm, deg_hbm, src_v, ones_v, zero_v, acc):
    c = lax.axis_index("c")
    s = lax.axis_index("s")
    wid = s * NC + c
    soff = pl.multiple_of(wid * PER_W, 8)
    pltpu.sync_copy(src_hbm.at[pl.ds(soff, PER_W)], src_v)

    one16 = jnp.ones((16,), jnp.float32)

    def fill_ones(i, _):
        ones_v[pl.ds(i * 16, 16)] = one16
        return 0

    lax.fori_loop(0, PER_W // 16, fill_ones, 0)

    @pl.when(s == 0)
    def _():
        zero16 = jnp.zeros((16,), jnp.float32)

        def fill_zero(i, _):
            zero_v[pl.ds(i * 16, 16)] = zero16
            return 0

        lax.fori_loop(0, NPAD // 16, fill_zero, 0)
        pltpu.sync_copy(zero_v, acc)

    plsc.subcore_barrier()
    pltpu.sync_copy(ones_v, acc.at[src_v], add=True)
    plsc.subcore_barrier()

    @pl.when(s == 0)
    def _():
        off = pl.multiple_of(c * NPAD, NPAD)
        pltpu.sync_copy(acc, deg_hbm.at[pl.ds(off, NPAD)])


def _deg_call(src_flat):
    k = functools.partial(
        pl.kernel,
        out_type=jax.ShapeDtypeStruct((NC * NPAD,), jnp.float32),
        mesh=_vsc_mesh(),
        scratch_types=[
            pltpu.VMEM((PER_W,), jnp.int32),
            pltpu.VMEM((PER_W,), jnp.float32),
            pltpu.VMEM((NPAD,), jnp.float32),
            pltpu.VMEM_SHARED((NPAD,), jnp.float32),
        ],
    )(_deg_body)
    return k(src_flat)


def _mm_body(x_ref, d0_ref, d1_ref, w_ref, y_ref):
    deg = d0_ref[...] + d1_ref[...]
    norm = lax.rsqrt(jnp.maximum(deg, 1.0))
    xs = x_ref[...] * norm
    y_ref[...] = lax.dot_general(
        xs, w_ref[...], (((1,), (1,)), ((), ())),
        preferred_element_type=jnp.float32,
    )


def _mm_call(x, deg_part, W):
    d0 = deg_part[:N].reshape(N, 1)
    d1 = deg_part[NPAD:NPAD + N].reshape(N, 1)
    return pl.pallas_call(
        _mm_body,
        grid=(NBLK,),
        in_specs=[
            pl.BlockSpec((BLK, D), lambda i: (i, 0)),
            pl.BlockSpec((BLK, 1), lambda i: (i, 0)),
            pl.BlockSpec((BLK, 1), lambda i: (i, 0)),
            pl.BlockSpec((D, D), lambda i: (0, 0)),
        ],
        out_specs=pl.BlockSpec((BLK, D), lambda i: (i, 0)),
        out_shape=jax.ShapeDtypeStruct((N, D), jnp.float32),
    )(x, d0, d1, W)


def _agg_body(
    y_hbm, src_hbm, dst_hbm, part_hbm, src_v, dst_v, rows_a, rows_b, rows_c,
    acc, gsem_a, gsem_b, gsem_c,
):
    c = lax.axis_index("c")
    s = lax.axis_index("s")
    wid = s * NC + c

    zero16 = jnp.zeros((16,), jnp.float32)

    def fill_zero(i, _):
        r = i // (D // 16)
        col = i % (D // 16)
        rows_a[r, pl.ds(col * 16, 16)] = zero16
        return 0

    lax.fori_loop(0, CH * (D // 16), fill_zero, 0)
    row0 = pl.multiple_of(s * RPT, 8)
    for kk in range(NRC):
        pltpu.async_copy(
            rows_a.at[pl.ds(0, RCH)], acc.at[pl.ds(row0 + kk * RCH, RCH)], gsem_a
        )

    @pl.when(s == NS - 1)
    def _():
        pltpu.async_copy(
            rows_a.at[pl.ds(0, TAIL)], acc.at[pl.ds(NS * RPT, TAIL)], gsem_b
        )

    for kk in range(NRC):
        pltpu.make_async_copy(
            rows_a.at[pl.ds(0, RCH)], acc.at[pl.ds(row0 + kk * RCH, RCH)], gsem_a
        ).wait()

    @pl.when(s == NS - 1)
    def _():
        pltpu.make_async_copy(
            rows_a.at[pl.ds(0, TAIL)], acc.at[pl.ds(NS * RPT, TAIL)], gsem_b
        ).wait()

    plsc.subcore_barrier()

    bufs = ((rows_a, gsem_a), (rows_b, gsem_b), (rows_c, gsem_c))
    for q in range(4):
        pltpu.sync_copy(src_hbm.at[wid, q], src_v)
        pltpu.sync_copy(dst_hbm.at[wid, q], dst_v)
        for p in range(3):
            pltpu.async_copy(y_hbm.at[src_v.at[p]], bufs[p][0], bufs[p][1])

        def body(i, _):
            for p in range(3):
                j = 3 * i + p
                rv, sm = bufs[p]
                pltpu.make_async_copy(y_hbm.at[src_v.at[j]], rv, sm).wait()
                pltpu.sync_copy(rv, acc.at[dst_v.at[j]], add=True)

                @pl.when(j + 3 < QF)
                def _():
                    pltpu.async_copy(y_hbm.at[src_v.at[j + 3]], rv, sm)

            return 0

        lax.fori_loop(0, QF // 3, body, 0)
        jt = QF - 1
        pltpu.make_async_copy(y_hbm.at[src_v.at[jt]], rows_a, gsem_a).wait()
        pltpu.sync_copy(rows_a, acc.at[dst_v.at[jt]], add=True)
    plsc.subcore_barrier()

    for kk in range(NRC):
        pltpu.async_copy(
            acc.at[pl.ds(row0 + kk * RCH, RCH)],
            part_hbm.at[c, pl.ds(row0 + kk * RCH, RCH)],
            gsem_a,
        )

    @pl.when(s == NS - 1)
    def _():
        pltpu.async_copy(
            acc.at[pl.ds(NS * RPT, TAIL)], part_hbm.at[c, pl.ds(NS * RPT, TAIL)],
            gsem_b,
        )

    for kk in range(NRC):
        pltpu.make_async_copy(
            acc.at[pl.ds(row0 + kk * RCH, RCH)],
            part_hbm.at[c, pl.ds(row0 + kk * RCH, RCH)],
            gsem_a,
        ).wait()

    @pl.when(s == NS - 1)
    def _():
        pltpu.make_async_copy(
            acc.at[pl.ds(NS * RPT, TAIL)], part_hbm.at[c, pl.ds(NS * RPT, TAIL)],
            gsem_b,
        ).wait()


def _agg_call(y, src3, dst3):
    src3 = src3.reshape(NW, 4, QF, CH)
    dst3 = dst3.reshape(NW, 4, QF, CH)
    k = functools.partial(
        pl.kernel,
        out_type=jax.ShapeDtypeStruct((NC, N, D), jnp.float32),
        mesh=_vsc_mesh(),
        scratch_types=[
            pltpu.VMEM((QF, CH), jnp.int32),
            pltpu.VMEM((QF, CH), jnp.int32),
            pltpu.VMEM((CH, D), jnp.float32),
            pltpu.VMEM((CH, D), jnp.float32),
            pltpu.VMEM((CH, D), jnp.float32),
            pltpu.VMEM_SHARED((N, D), jnp.float32),
            pltpu.SemaphoreType.DMA,
            pltpu.SemaphoreType.DMA,
            pltpu.SemaphoreType.DMA,
        ],
    )(_agg_body)
    return k(y, src3, dst3)


def _fin_body(p_ref, w_ref, b_ref, o_ref):
    bias2 = jnp.sum(w_ref[...], axis=1) + b_ref[0, :]
    o_ref[...] = p_ref[0] + p_ref[1] + bias2[None, :]


def _fin_call(part, W, b2):
    return pl.pallas_call(
        _fin_body,
        grid=(NBLK,),
        in_specs=[
            pl.BlockSpec((NC, BLK, D), lambda i: (0, i, 0)),
            pl.BlockSpec((D, D), lambda i: (0, 0)),
            pl.BlockSpec((1, D), lambda i: (0, 0)),
        ],
        out_specs=pl.BlockSpec((BLK, D), lambda i: (i, 0)),
        out_shape=jax.ShapeDtypeStruct((N, D), jnp.float32),
    )(part, W, b2)


def kernel(x, edge_index, W, b):
    src3 = edge_index[0].reshape(NW, NCH, CH)
    dst3 = edge_index[1].reshape(NW, NCH, CH)
    deg_part = _deg_call(edge_index[0])
    y = _mm_call(x, deg_part, W)
    part = _agg_call(y, src3, dst3)
    return _fin_call(part, W, b.reshape(1, D))
